# baseline (speedup 1.0000x reference)
import functools

import jax
import jax.numpy as jnp
from jax import lax
from jax.experimental import pallas as pl
from jax.experimental.pallas import tpu as pltpu
from jax.experimental.pallas import tpu_sc as plsc

BATCH = 16384
EMBED_DIM = 32
PACK = 4
WIDE = PACK * EMBED_DIM
NUM_CORES = 2
NUM_SUBCORES = 16
NUM_WORKERS = NUM_CORES * NUM_SUBCORES
B_PER_W = BATCH // NUM_WORKERS
CHUNK = 256
LANES = 16
REPACK_ROWS = 1000


def _repack_body(a_ref, b_ref, c_ref, d_ref, out_ref):
    out_ref[...] = jnp.concatenate(
        [a_ref[...], b_ref[...], c_ref[...], d_ref[...]], axis=1
    )


def _repack(table):
    num_rows = table.shape[0]
    quarter = num_rows // PACK
    nblk = quarter // REPACK_ROWS

    def spec(k):
        return pl.BlockSpec(
            (REPACK_ROWS, EMBED_DIM), lambda i, k=k: (i + k * nblk, 0)
        )

    return pl.pallas_call(
        _repack_body,
        grid=(nblk,),
        in_specs=[spec(0), spec(1), spec(2), spec(3)],
        out_specs=pl.BlockSpec((REPACK_ROWS, WIDE), lambda i: (i, 0)),
        out_shape=jax.ShapeDtypeStruct((quarter, WIDE), jnp.float32),
        compiler_params=pltpu.CompilerParams(
            dimension_semantics=(pltpu.ARBITRARY,)
        ),
    )(table, table, table, table)


def kernel(indices, table):
    idx = indices.astype(jnp.int32)
    quarter = table.shape[0] // PACK
    tabw = _repack(table)
    mesh = plsc.VectorSubcoreMesh(core_axis_name="c", subcore_axis_name="s")

    @functools.partial(
        pl.kernel,
        mesh=mesh,
        compiler_params=pltpu.CompilerParams(needs_layout_passes=False),
        out_type=jax.ShapeDtypeStruct((BATCH, EMBED_DIM), jnp.float32),
        scratch_types=[
            pltpu.VMEM((B_PER_W,), jnp.int32),
            pltpu.VMEM((B_PER_W,), jnp.int32),
            pltpu.VMEM((CHUNK, WIDE), jnp.float32),
            pltpu.VMEM((CHUNK, WIDE), jnp.float32),
            pltpu.VMEM((CHUNK, EMBED_DIM), jnp.float32),
            pltpu.SemaphoreType.DMA,
            pltpu.SemaphoreType.DMA,
            pltpu.SemaphoreType.DMA,
        ],
    )
    def gather_kernel(
        tab_hbm, idx_hbm, out_hbm,
        idx_v, q_v, rows_a, rows_b, out_v,
        sem_a, sem_b, sem_o,
    ):
        wid = lax.axis_index("s") * NUM_CORES + lax.axis_index("c")
        base = wid * B_PER_W
        pltpu.sync_copy(idx_hbm.at[pl.ds(base, B_PER_W)], idx_v)

        @pl.loop(0, B_PER_W, step=LANES)
        def _mod(i):
            v = idx_v[pl.ds(i, LANES)]
            rem = (
                jnp.where(v >= quarter, 1, 0)
                + jnp.where(v >= 2 * quarter, 1, 0)
                + jnp.where(v >= 3 * quarter, 1, 0)
            )
            q_v[pl.ds(i, LANES)] = v - rem * quarter

        ga = pltpu.async_copy(
            tab_hbm.at[q_v.at[pl.ds(0, CHUNK)]], rows_a, sem_a
        )
        gb = pltpu.async_copy(
            tab_hbm.at[q_v.at[pl.ds(CHUNK, CHUNK)]], rows_b, sem_b
        )

        def extract(o, rows_v):
            @pl.loop(0, CHUNK, step=LANES)
            def _group(j):
                iota = lax.iota(jnp.int32, LANES)
                v = idx_v[pl.ds(o + j, LANES)]
                rem = (
                    jnp.where(v >= quarter, 1, 0)
                    + jnp.where(v >= 2 * quarter, 1, 0)
                    + jnp.where(v >= 3 * quarter, 1, 0)
                )
                ibase = rem * EMBED_DIM
                jvec = iota + j
                czero = iota * 0
                for c in range(EMBED_DIM):
                    vals = plsc.load_gather(rows_v, [jvec, ibase + c])
                    plsc.store_scatter(out_v, [jvec, czero + c], vals)

        ga.wait()
        extract(0, rows_a)
        oa = pltpu.async_copy(out_v, out_hbm.at[pl.ds(base, CHUNK)], sem_o)
        gb.wait()
        oa.wait()
        extract(CHUNK, rows_b)
        pltpu.sync_copy(out_v, out_hbm.at[pl.ds(base + CHUNK, CHUNK)])

    return gather_kernel(tabw, idx)


# speedup vs baseline: 1.1855x; 1.1855x over previous
import functools

import jax
import jax.numpy as jnp
from jax import lax
from jax.experimental import pallas as pl
from jax.experimental.pallas import tpu as pltpu
from jax.experimental.pallas import tpu_sc as plsc

BATCH = 16384
EMBED_DIM = 32
PACK = 4
WIDE = PACK * EMBED_DIM
NUM_CORES = 2
NUM_SUBCORES = 16
NUM_WORKERS = NUM_CORES * NUM_SUBCORES
B_PER_W = BATCH // NUM_WORKERS
CHUNK = 256
LANES = 16
REPACK_ROWS = 5000


def _repack_body(a_ref, b_ref, c_ref, d_ref, out_ref):
    out_ref[...] = jnp.concatenate(
        [a_ref[...], b_ref[...], c_ref[...], d_ref[...]], axis=1
    )


def _repack(table):
    num_rows = table.shape[0]
    quarter = num_rows // PACK
    nblk = quarter // REPACK_ROWS

    def spec(k):
        return pl.BlockSpec(
            (REPACK_ROWS, EMBED_DIM), lambda i, k=k: (i + k * nblk, 0)
        )

    return pl.pallas_call(
        _repack_body,
        grid=(nblk,),
        in_specs=[spec(0), spec(1), spec(2), spec(3)],
        out_specs=pl.BlockSpec((REPACK_ROWS, WIDE), lambda i: (i, 0)),
        out_shape=jax.ShapeDtypeStruct((quarter, WIDE), jnp.float32),
    )(table, table, table, table)


def kernel(indices, table):
    idx = indices.astype(jnp.int32)
    quarter = table.shape[0] // PACK
    tabw = _repack(table)
    mesh = plsc.VectorSubcoreMesh(core_axis_name="c", subcore_axis_name="s")

    @functools.partial(
        pl.kernel,
        mesh=mesh,
        compiler_params=pltpu.CompilerParams(needs_layout_passes=False),
        out_type=jax.ShapeDtypeStruct((BATCH, EMBED_DIM), jnp.float32),
        scratch_types=[
            pltpu.VMEM((B_PER_W,), jnp.int32),
            pltpu.VMEM((B_PER_W,), jnp.int32),
            pltpu.VMEM((CHUNK, WIDE), jnp.float32),
            pltpu.VMEM((CHUNK, WIDE), jnp.float32),
            pltpu.VMEM((CHUNK, EMBED_DIM), jnp.float32),
            pltpu.SemaphoreType.DMA,
            pltpu.SemaphoreType.DMA,
            pltpu.SemaphoreType.DMA,
        ],
    )
    def gather_kernel(
        tab_hbm, idx_hbm, out_hbm,
        idx_v, q_v, rows_a, rows_b, out_v,
        sem_a, sem_b, sem_o,
    ):
        wid = lax.axis_index("s") * NUM_CORES + lax.axis_index("c")
        base = wid * B_PER_W
        pltpu.sync_copy(idx_hbm.at[pl.ds(base, B_PER_W)], idx_v)

        @pl.loop(0, B_PER_W, step=LANES)
        def _mod(i):
            v = idx_v[pl.ds(i, LANES)]
            rem = (
                jnp.where(v >= quarter, 1, 0)
                + jnp.where(v >= 2 * quarter, 1, 0)
                + jnp.where(v >= 3 * quarter, 1, 0)
            )
            q_v[pl.ds(i, LANES)] = v - rem * quarter

        ga = pltpu.async_copy(
            tab_hbm.at[q_v.at[pl.ds(0, CHUNK)]], rows_a, sem_a
        )
        gb = pltpu.async_copy(
            tab_hbm.at[q_v.at[pl.ds(CHUNK, CHUNK)]], rows_b, sem_b
        )

        def extract(o, rows_v):
            @pl.loop(0, CHUNK, step=LANES)
            def _group(j):
                iota = lax.iota(jnp.int32, LANES)
                v = idx_v[pl.ds(o + j, LANES)]
                rem = (
                    jnp.where(v >= quarter, 1, 0)
                    + jnp.where(v >= 2 * quarter, 1, 0)
                    + jnp.where(v >= 3 * quarter, 1, 0)
                )
                ibase = rem * EMBED_DIM
                jvec = iota + j
                czero = iota * 0
                for c in range(EMBED_DIM):
                    vals = plsc.load_gather(rows_v, [jvec, ibase + c])
                    plsc.store_scatter(out_v, [jvec, czero + c], vals)

        ga.wait()
        extract(0, rows_a)
        oa = pltpu.async_copy(out_v, out_hbm.at[pl.ds(base, CHUNK)], sem_o)
        gb.wait()
        oa.wait()
        extract(CHUNK, rows_b)
        pltpu.sync_copy(out_v, out_hbm.at[pl.ds(base + CHUNK, CHUNK)])

    return gather_kernel(tabw, idx)


# speedup vs baseline: 1.2092x; 1.0200x over previous
import functools

import jax
import jax.numpy as jnp
from jax import lax
from jax.experimental import pallas as pl
from jax.experimental.pallas import tpu as pltpu
from jax.experimental.pallas import tpu_sc as plsc

BATCH = 16384
EMBED_DIM = 32
NUM_CORES = 2
NUM_SUBCORES = 16
NUM_WORKERS = NUM_CORES * NUM_SUBCORES
B_PER_W = BATCH // NUM_WORKERS


def kernel(indices, table):
    idx = indices.astype(jnp.int32)
    mesh = plsc.VectorSubcoreMesh(core_axis_name="c", subcore_axis_name="s")

    @functools.partial(
        pl.kernel,
        mesh=mesh,
        compiler_params=pltpu.CompilerParams(use_tc_tiling_on_sc=False),
        out_type=jax.ShapeDtypeStruct((BATCH, EMBED_DIM), jnp.float32),
        scratch_types=[
            pltpu.VMEM((B_PER_W,), jnp.int32),
            pltpu.VMEM((B_PER_W, EMBED_DIM), jnp.float32),
            pltpu.SemaphoreType.DMA,
        ],
    )
    def gather_kernel(tab_hbm, idx_hbm, out_hbm, idx_v, rows_v, sem):
        wid = lax.axis_index("s") * NUM_CORES + lax.axis_index("c")
        base = wid * B_PER_W
        pltpu.sync_copy(idx_hbm.at[pl.ds(base, B_PER_W)], idx_v)
        pltpu.async_copy(tab_hbm.at[idx_v], rows_v, sem).wait()
        pltpu.sync_copy(rows_v, out_hbm.at[pl.ds(base, B_PER_W)])

    return gather_kernel(table, idx)


# speedup vs baseline: 1.2479x; 1.0320x over previous
import functools

import jax
import jax.numpy as jnp
from jax import lax
from jax.experimental import pallas as pl
from jax.experimental.pallas import tpu as pltpu
from jax.experimental.pallas import tpu_sc as plsc

BATCH = 16384
EMBED_DIM = 32
WIDE = 128
NUM_CORES = 2
NUM_SUBCORES = 16
NUM_WORKERS = NUM_CORES * NUM_SUBCORES
B_PER_W = BATCH // NUM_WORKERS


def kernel(indices, table):
    idx = indices.astype(jnp.int32)
    tabp = jnp.pad(table, ((0, 0), (0, WIDE - EMBED_DIM)))
    mesh = plsc.VectorSubcoreMesh(core_axis_name="c", subcore_axis_name="s")

    @functools.partial(
        pl.kernel,
        mesh=mesh,
        out_type=jax.ShapeDtypeStruct((BATCH, WIDE), jnp.float32),
        scratch_types=[
            pltpu.VMEM((B_PER_W,), jnp.int32),
            pltpu.VMEM((B_PER_W, WIDE), jnp.float32),
            pltpu.SemaphoreType.DMA,
        ],
    )
    def gather_kernel(tab_hbm, idx_hbm, out_hbm, idx_v, rows_v, sem):
        wid = lax.axis_index("s") * NUM_CORES + lax.axis_index("c")
        base = wid * B_PER_W
        pltpu.sync_copy(idx_hbm.at[pl.ds(base, B_PER_W)], idx_v)
        pltpu.async_copy(tab_hbm.at[idx_v], rows_v, sem).wait()
        pltpu.sync_copy(rows_v, out_hbm.at[pl.ds(base, B_PER_W)])

    out_wide = gather_kernel(tabp, idx)
    return out_wide[:, :EMBED_DIM]
